# Initial kernel scaffold; baseline (speedup 1.0000x reference)
import functools

import jax
import jax.numpy as jnp
from jax import lax
from jax.experimental import pallas as pl
from jax.experimental.pallas import tpu as pltpu
from jax.experimental.pallas import tpu_sc as plsc

D = 32
P = 20
NNEG = 50
J = P + NNEG
JPAD = 80
NLANE = 16
NWORKERS = 32
GROUP = 16


def _sc_scores(c_word, idx_all, i_emb, o_emb):
    B = c_word.shape[0]
    nb_per_w = B // NWORKERS
    ngroups = nb_per_w // GROUP

    mesh = plsc.VectorSubcoreMesh(core_axis_name="c", subcore_axis_name="s")

    @functools.partial(
        pl.kernel,
        out_type=jax.ShapeDtypeStruct((B, JPAD), jnp.float32),
        mesh=mesh,
        scratch_types=[
            pltpu.VMEM((2, GROUP, J), jnp.int32),
            pltpu.VMEM((2, GROUP), jnp.int32),
            pltpu.VMEM((2, GROUP, D), jnp.float32),
            pltpu.VMEM((2, GROUP, J, D), jnp.float32),
            pltpu.VMEM((2, GROUP, JPAD), jnp.float32),
            pltpu.SemaphoreType.DMA,
            pltpu.SemaphoreType.DMA,
            pltpu.SemaphoreType.DMA,
            pltpu.SemaphoreType.DMA,
        ],
    )
    def sc_kernel(c_word_hbm, idx_hbm, i_emb_hbm, o_emb_hbm, out_hbm,
                  idxs_v, cidx_v, crows_v, rows_v, scores_v,
                  sem_rows0, sem_rows1, sem_out0, sem_out1):
        ncores = 2
        wid = lax.axis_index("s") * ncores + lax.axis_index("c")
        base_b = wid * nb_per_w

        lanes = lax.broadcasted_iota(jnp.int32, (NLANE,), 0)
        tail_mask = lanes < (J - 4 * NLANE)
        sems_rows = (sem_rows0, sem_rows1)
        sems_out = (sem_out0, sem_out1)

        def fetch_group(g, slot):
            b0 = base_b + g * GROUP
            pltpu.sync_copy(idx_hbm.at[pl.ds(b0, GROUP)], idxs_v.at[slot])
            pltpu.sync_copy(c_word_hbm.at[pl.ds(b0, GROUP)], cidx_v.at[slot])
            pltpu.async_copy(i_emb_hbm.at[cidx_v.at[slot]],
                             crows_v.at[slot], sems_rows[slot])
            for bi in range(GROUP):
                pltpu.async_copy(o_emb_hbm.at[idxs_v.at[slot, bi]],
                                 rows_v.at[slot, bi], sems_rows[slot])

        def drain_group(slot):
            pltpu.make_async_copy(i_emb_hbm.at[cidx_v.at[slot]],
                                  crows_v.at[slot], sems_rows[slot]).wait()
            for bi in range(GROUP):
                pltpu.make_async_copy(o_emb_hbm.at[idxs_v.at[slot, bi]],
                                      rows_v.at[slot, bi],
                                      sems_rows[slot]).wait()

        def compute_group(g, slot):
            drain_group(slot)

            def b_body(bi, _):
                accs = [jnp.zeros((NLANE,), jnp.float32) for _ in range(5)]
                bsplat = jnp.broadcast_to(bi, (NLANE,)).astype(jnp.int32)
                for d in range(D):
                    cd = crows_v[slot, bi, d]
                    cdv = jnp.broadcast_to(cd, (NLANE,))
                    dsplat = jnp.full((NLANE,), d, jnp.int32)
                    for t in range(5):
                        jidx = lanes + t * NLANE
                        if t < 4:
                            col = plsc.load_gather(
                                rows_v.at[slot], [bsplat, jidx, dsplat])
                        else:
                            col = plsc.load_gather(
                                rows_v.at[slot], [bsplat, jidx, dsplat],
                                mask=tail_mask)
                        accs[t] = accs[t] + col * cdv
                for t in range(5):
                    scores_v[slot, bi, pl.ds(t * NLANE, NLANE)] = accs[t]
                return 0

            lax.fori_loop(0, GROUP, b_body, 0)
            b0 = base_b + g * GROUP
            pltpu.async_copy(scores_v.at[slot],
                             out_hbm.at[pl.ds(b0, GROUP)], sems_out[slot])

        def drain_out(g, slot):
            b0 = base_b + g * GROUP
            pltpu.make_async_copy(scores_v.at[slot],
                                  out_hbm.at[pl.ds(b0, GROUP)],
                                  sems_out[slot]).wait()

        fetch_group(0, 0)

        def pair_body(i, _):
            g0 = 2 * i
            fetch_group(g0 + 1, 1)

            @pl.when(i > 0)
            def _():
                drain_out(g0 - 2, 0)
            compute_group(g0, 0)

            @pl.when(g0 + 2 < ngroups)
            def _():
                fetch_group(g0 + 2, 0)

            @pl.when(i > 0)
            def _():
                drain_out(g0 - 1, 1)
            compute_group(g0 + 1, 1)
            return 0

        lax.fori_loop(0, ngroups // 2, pair_body, 0)
        drain_out(ngroups - 2, 0)
        drain_out(ngroups - 1, 1)

    return sc_kernel(c_word, idx_all, i_emb, o_emb)


def _tc_loss(scores):
    B = scores.shape[0]
    BLK = 512

    def tc_kernel(s_ref, o_ref):
        x = s_ref[...]
        jcol = lax.broadcasted_iota(jnp.int32, x.shape, 1)
        z = jnp.where(jcol < P, x, -x)
        ls = jax.nn.log_sigmoid(z)
        ls = jnp.where(jcol < J, ls, 0.0)
        o_ref[...] = -jnp.sum(ls, axis=1, keepdims=True)

    out = pl.pallas_call(
        tc_kernel,
        out_shape=jax.ShapeDtypeStruct((B, 1), jnp.float32),
        grid=(B // BLK,),
        in_specs=[pl.BlockSpec((BLK, JPAD), lambda i: (i, 0))],
        out_specs=pl.BlockSpec((BLK, 1), lambda i: (i, 0)),
    )(scores)
    return out.reshape(B)


@jax.jit
def kernel(c_word, bg_word_pos, bg_word_neg, i_emb, o_emb):
    idx_all = jnp.concatenate(
        [bg_word_pos.astype(jnp.int32), bg_word_neg.astype(jnp.int32)],
        axis=1)
    scores = _sc_scores(c_word.astype(jnp.int32), idx_all, i_emb, o_emb)
    return _tc_loss(scores)


# speedup vs baseline: 1.5035x; 1.5035x over previous
import functools

import jax
import jax.numpy as jnp
from jax import lax
from jax.experimental import pallas as pl
from jax.experimental.pallas import tpu as pltpu
from jax.experimental.pallas import tpu_sc as plsc

D = 32
P = 20
NNEG = 50
J = P + NNEG
JPAD = 80
NLANE = 16
NWORKERS = 32
GROUP = 16


def _sc_scores(c_word, idx_all, i_emb, o_emb):
    B = c_word.shape[0]
    nb_per_w = B // NWORKERS
    ngroups = nb_per_w // GROUP

    mesh = plsc.VectorSubcoreMesh(core_axis_name="c", subcore_axis_name="s")

    @functools.partial(
        pl.kernel,
        out_type=jax.ShapeDtypeStruct((B, JPAD), jnp.float32),
        mesh=mesh,
        compiler_params=pltpu.CompilerParams(
            needs_layout_passes=False, use_tc_tiling_on_sc=False),
        scratch_types=[
            pltpu.VMEM((2, GROUP, J), jnp.int32),
            pltpu.VMEM((2, GROUP), jnp.int32),
            pltpu.VMEM((2, GROUP, D), jnp.float32),
            pltpu.VMEM((2 * GROUP * J, D), jnp.float32),
            pltpu.VMEM((2, GROUP, JPAD), jnp.float32),
            pltpu.SemaphoreType.DMA,
            pltpu.SemaphoreType.DMA,
            pltpu.SemaphoreType.DMA,
            pltpu.SemaphoreType.DMA,
        ],
    )
    def sc_kernel(c_word_hbm, idx_hbm, i_emb_hbm, o_emb_hbm, out_hbm,
                  idxs_v, cidx_v, crows_v, rows_v, scores_v,
                  sem_rows0, sem_rows1, sem_out0, sem_out1):
        ncores = 2
        wid = lax.axis_index("s") * ncores + lax.axis_index("c")
        base_b = wid * nb_per_w

        lanes = lax.broadcasted_iota(jnp.int32, (NLANE,), 0)
        tail_mask = lanes < (J - 4 * NLANE)
        sems_rows = (sem_rows0, sem_rows1)
        sems_out = (sem_out0, sem_out1)

        def fetch_group(g, slot):
            b0 = base_b + g * GROUP
            pltpu.sync_copy(idx_hbm.at[pl.ds(b0, GROUP)], idxs_v.at[slot])
            pltpu.sync_copy(c_word_hbm.at[pl.ds(b0, GROUP)], cidx_v.at[slot])
            pltpu.async_copy(i_emb_hbm.at[cidx_v.at[slot]],
                             crows_v.at[slot], sems_rows[slot])
            for bi in range(GROUP):
                pltpu.async_copy(
                    o_emb_hbm.at[idxs_v.at[slot, bi]],
                    rows_v.at[pl.ds((slot * GROUP + bi) * J, J)],
                    sems_rows[slot])

        def drain_group(slot):
            pltpu.make_async_copy(i_emb_hbm.at[cidx_v.at[slot]],
                                  crows_v.at[slot], sems_rows[slot]).wait()
            for bi in range(GROUP):
                pltpu.make_async_copy(
                    o_emb_hbm.at[idxs_v.at[slot, bi]],
                    rows_v.at[pl.ds((slot * GROUP + bi) * J, J)],
                    sems_rows[slot]).wait()

        def compute_group(g, slot):
            drain_group(slot)

            def b_body(bi, _):
                accs = [jnp.zeros((NLANE,), jnp.float32) for _ in range(5)]
                bsplat = jnp.broadcast_to(bi, (NLANE,)).astype(jnp.int32)
                chalves = [crows_v[slot, bi, pl.ds(0, NLANE)],
                           crows_v[slot, bi, pl.ds(NLANE, NLANE)]]
                rbase = (jnp.broadcast_to(bi, (NLANE,)).astype(jnp.int32)
                         * J + slot * GROUP * J)
                for d in range(D):
                    cd = chalves[d // NLANE][d % NLANE]
                    cdv = jnp.broadcast_to(cd, (NLANE,))
                    dsplat = jnp.full((NLANE,), d, jnp.int32)
                    for t in range(5):
                        ridx = rbase + (lanes + t * NLANE)
                        if t < 4:
                            col = plsc.load_gather(
                                rows_v, [ridx, dsplat])
                        else:
                            col = plsc.load_gather(
                                rows_v, [ridx, dsplat],
                                mask=tail_mask)
                        accs[t] = accs[t] + col * cdv
                for t in range(5):
                    scores_v[slot, bi, pl.ds(t * NLANE, NLANE)] = accs[t]
                return 0

            lax.fori_loop(0, GROUP, b_body, 0)
            b0 = base_b + g * GROUP
            pltpu.async_copy(scores_v.at[slot],
                             out_hbm.at[pl.ds(b0, GROUP)], sems_out[slot])

        def drain_out(g, slot):
            b0 = base_b + g * GROUP
            pltpu.make_async_copy(scores_v.at[slot],
                                  out_hbm.at[pl.ds(b0, GROUP)],
                                  sems_out[slot]).wait()

        fetch_group(0, 0)

        def pair_body(i, _):
            g0 = 2 * i
            fetch_group(g0 + 1, 1)

            @pl.when(i > 0)
            def _():
                drain_out(g0 - 2, 0)
            compute_group(g0, 0)

            @pl.when(g0 + 2 < ngroups)
            def _():
                fetch_group(g0 + 2, 0)

            @pl.when(i > 0)
            def _():
                drain_out(g0 - 1, 1)
            compute_group(g0 + 1, 1)
            return 0

        lax.fori_loop(0, ngroups // 2, pair_body, 0)
        drain_out(ngroups - 2, 0)
        drain_out(ngroups - 1, 1)

    return sc_kernel(c_word, idx_all, i_emb, o_emb)


def _tc_loss(scores):
    B = scores.shape[0]
    BLK = 512

    def tc_kernel(s_ref, o_ref):
        x = s_ref[...]
        jcol = lax.broadcasted_iota(jnp.int32, x.shape, 1)
        z = jnp.where(jcol < P, x, -x)
        ls = jax.nn.log_sigmoid(z)
        ls = jnp.where(jcol < J, ls, 0.0)
        o_ref[...] = -jnp.sum(ls, axis=1, keepdims=True)

    out = pl.pallas_call(
        tc_kernel,
        out_shape=jax.ShapeDtypeStruct((B, 1), jnp.float32),
        grid=(B // BLK,),
        in_specs=[pl.BlockSpec((BLK, JPAD), lambda i: (i, 0))],
        out_specs=pl.BlockSpec((BLK, 1), lambda i: (i, 0)),
    )(scores)
    return out.reshape(B)


@jax.jit
def kernel(c_word, bg_word_pos, bg_word_neg, i_emb, o_emb):
    idx_all = jnp.concatenate(
        [bg_word_pos.astype(jnp.int32), bg_word_neg.astype(jnp.int32)],
        axis=1)
    scores = _sc_scores(c_word.astype(jnp.int32), idx_all, i_emb, o_emb)
    return _tc_loss(scores)


# speedup vs baseline: 1.6956x; 1.1277x over previous
import functools

import jax
import jax.numpy as jnp
from jax import lax
from jax.experimental import pallas as pl
from jax.experimental.pallas import tpu as pltpu
from jax.experimental.pallas import tpu_sc as plsc

D = 32
P = 20
NNEG = 50
J = P + NNEG
JP = 72
JPAD = 80
NLANE = 16
NWORKERS = 32
GROUP = 8


def _sc_scores(c_word, idx_flat, i_emb, o_emb):
    B = c_word.shape[0]
    nb_per_w = B // NWORKERS
    ngroups = nb_per_w // GROUP

    mesh = plsc.VectorSubcoreMesh(core_axis_name="c", subcore_axis_name="s")

    @functools.partial(
        pl.kernel,
        out_type=jax.ShapeDtypeStruct((B * JPAD,), jnp.float32),
        mesh=mesh,
        compiler_params=pltpu.CompilerParams(
            needs_layout_passes=False, use_tc_tiling_on_sc=False),
        scratch_types=[
            pltpu.VMEM((2 * GROUP * JP,), jnp.int32),
            pltpu.VMEM((2 * GROUP,), jnp.int32),
            pltpu.VMEM((2 * GROUP, D), jnp.float32),
            pltpu.VMEM((2 * GROUP * JP, D), jnp.float32),
            pltpu.VMEM((2 * GROUP * JPAD,), jnp.float32),
            pltpu.SemaphoreType.DMA,
            pltpu.SemaphoreType.DMA,
            pltpu.SemaphoreType.DMA,
            pltpu.SemaphoreType.DMA,
        ],
    )
    def sc_kernel(c_word_hbm, idx_hbm, i_emb_hbm, o_emb_hbm, out_hbm,
                  idxs_v, cidx_v, crows_v, rows_v, scores_v,
                  sem_rows0, sem_rows1, sem_out0, sem_out1):
        ncores = 2
        wid = lax.axis_index("s") * ncores + lax.axis_index("c")
        base_b = wid * nb_per_w

        lanes = lax.broadcasted_iota(jnp.int32, (NLANE,), 0)
        tail_mask = lanes < (J - 4 * NLANE)
        lt = [lanes + t * NLANE for t in range(5)]
        sems_rows = (sem_rows0, sem_rows1)
        sems_out = (sem_out0, sem_out1)

        def fetch_group(g, slot):
            b0 = base_b + g * GROUP
            pltpu.sync_copy(idx_hbm.at[pl.ds(b0 * JP, GROUP * JP)],
                            idxs_v.at[pl.ds(slot * GROUP * JP, GROUP * JP)])
            pltpu.sync_copy(c_word_hbm.at[pl.ds(b0, GROUP)],
                            cidx_v.at[pl.ds(slot * GROUP, GROUP)])
            pltpu.async_copy(i_emb_hbm.at[cidx_v.at[pl.ds(slot * GROUP,
                                                          GROUP)]],
                             crows_v.at[pl.ds(slot * GROUP, GROUP)],
                             sems_rows[slot])
            for bi in range(GROUP):
                r0 = (slot * GROUP + bi) * JP
                pltpu.async_copy(o_emb_hbm.at[idxs_v.at[pl.ds(r0, JP)]],
                                 rows_v.at[pl.ds(r0, JP)], sems_rows[slot])

        def drain_group(slot):
            pltpu.make_async_copy(
                i_emb_hbm.at[cidx_v.at[pl.ds(slot * GROUP, GROUP)]],
                crows_v.at[pl.ds(slot * GROUP, GROUP)],
                sems_rows[slot]).wait()
            for bi in range(GROUP):
                r0 = (slot * GROUP + bi) * JP
                pltpu.make_async_copy(o_emb_hbm.at[idxs_v.at[pl.ds(r0, JP)]],
                                      rows_v.at[pl.ds(r0, JP)],
                                      sems_rows[slot]).wait()

        def compute_group(g, slot):
            drain_group(slot)

            def b_body(bi, _):
                accs = [jnp.zeros((NLANE,), jnp.float32) for _ in range(5)]
                row_i = jnp.broadcast_to(bi + slot * GROUP,
                                         (NLANE,)).astype(jnp.int32)
                rbase = row_i * JP
                for d in range(D):
                    dvec = (lanes + d) & (D - 1)
                    cdv = plsc.load_gather(crows_v, [row_i, dvec])
                    for t in range(5):
                        ridx = rbase + lt[t]
                        if t < 4:
                            col = plsc.load_gather(rows_v, [ridx, dvec])
                        else:
                            col = plsc.load_gather(rows_v, [ridx, dvec],
                                                   mask=tail_mask)
                        accs[t] = accs[t] + col * cdv
                s0 = (slot * GROUP + bi) * JPAD
                for t in range(5):
                    scores_v[pl.ds(s0 + t * NLANE, NLANE)] = accs[t]
                return 0

            lax.fori_loop(0, GROUP, b_body, 0)
            b0 = base_b + g * GROUP
            pltpu.async_copy(
                scores_v.at[pl.ds(slot * GROUP * JPAD, GROUP * JPAD)],
                out_hbm.at[pl.ds(b0 * JPAD, GROUP * JPAD)], sems_out[slot])

        def drain_out(g, slot):
            b0 = base_b + g * GROUP
            pltpu.make_async_copy(
                scores_v.at[pl.ds(slot * GROUP * JPAD, GROUP * JPAD)],
                out_hbm.at[pl.ds(b0 * JPAD, GROUP * JPAD)],
                sems_out[slot]).wait()

        fetch_group(0, 0)

        def pair_body(i, _):
            g0 = 2 * i
            fetch_group(g0 + 1, 1)

            @pl.when(i > 0)
            def _():
                drain_out(g0 - 2, 0)
            compute_group(g0, 0)

            @pl.when(g0 + 2 < ngroups)
            def _():
                fetch_group(g0 + 2, 0)

            @pl.when(i > 0)
            def _():
                drain_out(g0 - 1, 1)
            compute_group(g0 + 1, 1)
            return 0

        lax.fori_loop(0, ngroups // 2, pair_body, 0)
        drain_out(ngroups - 2, 0)
        drain_out(ngroups - 1, 1)

    return sc_kernel(c_word, idx_flat, i_emb, o_emb)


def _tc_loss(scores):
    B = scores.shape[0]
    BLK = 512

    def tc_kernel(s_ref, o_ref):
        x = s_ref[...]
        jcol = lax.broadcasted_iota(jnp.int32, x.shape, 1)
        z = jnp.where(jcol < P, x, -x)
        ls = jax.nn.log_sigmoid(z)
        ls = jnp.where(jcol < J, ls, 0.0)
        o_ref[...] = -jnp.sum(ls, axis=1, keepdims=True)

    out = pl.pallas_call(
        tc_kernel,
        out_shape=jax.ShapeDtypeStruct((B, 1), jnp.float32),
        grid=(B // BLK,),
        in_specs=[pl.BlockSpec((BLK, JPAD), lambda i: (i, 0))],
        out_specs=pl.BlockSpec((BLK, 1), lambda i: (i, 0)),
    )(scores)
    return out.reshape(B)


@jax.jit
def kernel(c_word, bg_word_pos, bg_word_neg, i_emb, o_emb):
    B = c_word.shape[0]
    idx_all = jnp.concatenate(
        [bg_word_pos.astype(jnp.int32), bg_word_neg.astype(jnp.int32),
         jnp.zeros((B, JP - J), jnp.int32)], axis=1).reshape(-1)
    scores = _sc_scores(c_word.astype(jnp.int32), idx_all, i_emb, o_emb)
    return _tc_loss(scores.reshape(B, JPAD))
